# Initial kernel scaffold; baseline (speedup 1.0000x reference)
import jax
import jax.numpy as jnp
from jax.experimental import pallas as pl


def kernel(x, gate_W, gate_b, temperature, expert_W, expert_b):
    raise NotImplementedError("write your pallas kernel here")


# speedup vs baseline: 1.9922x; 1.9922x over previous
import jax
import jax.numpy as jnp
from jax.experimental import pallas as pl
from jax.experimental.pallas import tpu as pltpu

E = 8
D = 1024
TN = 512


def _body(x_ref, gw_ref, gb_ref, temp_ref, ew_ref, eb_ref, out_ref, mask_ref):
    e = pl.program_id(1)

    @pl.when(e == 0)
    def _():
        scores = jax.lax.dot_general(
            x_ref[...], gw_ref[...], (((1,), (1,)), ((), ())),
            preferred_element_type=jnp.float32)
        scores = (scores + gb_ref[...]) / temp_ref[0, 0]
        iota = jax.lax.broadcasted_iota(jnp.int32, (TN, E), 1)
        neg = jnp.float32(-jnp.inf)
        vmax1 = jnp.max(scores, axis=1, keepdims=True)
        a1 = jnp.min(jnp.where(scores == vmax1, iota, E), axis=1, keepdims=True)
        s2 = jnp.where(iota == a1, neg, scores)
        vmax2 = jnp.max(s2, axis=1, keepdims=True)
        a2 = jnp.min(jnp.where(s2 == vmax2, iota, E), axis=1, keepdims=True)
        mask_ref[...] = ((iota == a1) | (iota == a2)).astype(jnp.float32)

    contrib = jax.lax.dot_general(
        x_ref[...], ew_ref[0], (((1,), (0,)), ((), ())),
        preferred_element_type=jnp.float32) + eb_ref[0]
    iota = jax.lax.broadcasted_iota(jnp.int32, (TN, E), 1)
    m = jnp.sum(mask_ref[...] * (iota == e).astype(jnp.float32), axis=1,
                keepdims=True)
    contrib = contrib * m

    @pl.when(e == 0)
    def _():
        out_ref[...] = contrib

    @pl.when(e != 0)
    def _():
        out_ref[...] += contrib


def kernel(x, gate_W, gate_b, temperature, expert_W, expert_b):
    N = x.shape[0]
    grid = (N // TN, E)
    out = pl.pallas_call(
        _body,
        grid=grid,
        in_specs=[
            pl.BlockSpec((TN, D), lambda t, e: (t, 0)),
            pl.BlockSpec((E, D), lambda t, e: (0, 0)),
            pl.BlockSpec((1, E), lambda t, e: (0, 0)),
            pl.BlockSpec((1, 1), lambda t, e: (0, 0)),
            pl.BlockSpec((1, D, D), lambda t, e: (e, 0, 0)),
            pl.BlockSpec((1, 1, D), lambda t, e: (e, 0, 0)),
        ],
        out_specs=pl.BlockSpec((TN, D), lambda t, e: (t, 0)),
        out_shape=jax.ShapeDtypeStruct((N, D), jnp.float32),
        scratch_shapes=[pltpu.VMEM((TN, E), jnp.float32)],
    )(
        x, gate_W, gate_b.reshape(1, E), temperature.reshape(1, 1),
        expert_W, expert_b.reshape(E, 1, D),
    )
    return out[:, None, :]


# speedup vs baseline: 1.9942x; 1.0010x over previous
import jax
import jax.numpy as jnp
from jax.experimental import pallas as pl
from jax.experimental.pallas import tpu as pltpu

E = 8
D = 1024
TN = 512


def _body(x_ref, gw_ref, gb_ref, temp_ref, ew_ref, eb_ref, out_ref, mask_ref):
    e = pl.program_id(1)

    @pl.when(e == 0)
    def _():
        scores = jax.lax.dot_general(
            x_ref[...], gw_ref[...], (((1,), (1,)), ((), ())),
            preferred_element_type=jnp.float32)
        scores = (scores + gb_ref[...]) / temp_ref[0, 0]
        iota = jax.lax.broadcasted_iota(jnp.int32, (TN, E), 1)
        neg = jnp.float32(-jnp.inf)
        vmax1 = jnp.max(scores, axis=1, keepdims=True)
        a1 = jnp.min(jnp.where(scores == vmax1, iota, E), axis=1, keepdims=True)
        s2 = jnp.where(iota == a1, neg, scores)
        vmax2 = jnp.max(s2, axis=1, keepdims=True)
        a2 = jnp.min(jnp.where(s2 == vmax2, iota, E), axis=1, keepdims=True)
        mask_ref[...] = ((iota == a1) | (iota == a2)).astype(jnp.float32)

    contrib = jax.lax.dot_general(
        x_ref[...].astype(jnp.bfloat16), ew_ref[0].astype(jnp.bfloat16),
        (((1,), (0,)), ((), ())),
        preferred_element_type=jnp.float32) + eb_ref[0]
    iota = jax.lax.broadcasted_iota(jnp.int32, (TN, E), 1)
    m = jnp.sum(mask_ref[...] * (iota == e).astype(jnp.float32), axis=1,
                keepdims=True)
    contrib = contrib * m

    @pl.when(e == 0)
    def _():
        out_ref[...] = contrib

    @pl.when(e != 0)
    def _():
        out_ref[...] += contrib


def kernel(x, gate_W, gate_b, temperature, expert_W, expert_b):
    N = x.shape[0]
    grid = (N // TN, E)
    out = pl.pallas_call(
        _body,
        grid=grid,
        in_specs=[
            pl.BlockSpec((TN, D), lambda t, e: (t, 0)),
            pl.BlockSpec((E, D), lambda t, e: (0, 0)),
            pl.BlockSpec((1, E), lambda t, e: (0, 0)),
            pl.BlockSpec((1, 1), lambda t, e: (0, 0)),
            pl.BlockSpec((1, D, D), lambda t, e: (e, 0, 0)),
            pl.BlockSpec((1, 1, D), lambda t, e: (e, 0, 0)),
        ],
        out_specs=pl.BlockSpec((TN, D), lambda t, e: (t, 0)),
        out_shape=jax.ShapeDtypeStruct((N, D), jnp.float32),
        scratch_shapes=[pltpu.VMEM((TN, E), jnp.float32)],
    )(
        x, gate_W, gate_b.reshape(1, E), temperature.reshape(1, 1),
        expert_W, expert_b.reshape(E, 1, D),
    )
    return out[:, None, :]
